# baseline (speedup 1.0000x reference)
import functools

import jax
import jax.numpy as jnp
from jax import lax
from jax.experimental import pallas as pl
from jax.experimental.pallas import tpu as pltpu
from jax.experimental.pallas import tpu_sc as plsc

NUM_IDS = 131072
NUM_BINS = 256
PACKET = 512
NC = 2
NS = 16
LANES = 16
NW = NC * NS
SC_N = NUM_IDS // 2
PER_W = SC_N // NW
CHUNKS = PER_W // LANES
TC_ROWS = (NUM_IDS - SC_N) // 128
TC_BLOCK = 32


@functools.lru_cache(maxsize=None)
def _make_sc_hist():
    mesh = plsc.VectorSubcoreMesh(
        core_axis_name="c", subcore_axis_name="s", num_cores=NC, num_subcores=NS
    )

    @functools.partial(
        pl.kernel,
        out_type=jax.ShapeDtypeStruct((NW, LANES * NUM_BINS), jnp.float32),
        mesh=mesh,
        scratch_types=[
            pltpu.VMEM((PER_W,), jnp.int32),
            pltpu.VMEM((LANES * NUM_BINS,), jnp.float32),
            pltpu.SemaphoreType.DMA,
        ],
        compiler_params=pltpu.CompilerParams(needs_layout_passes=False),
    )
    def _sc_hist(ids_hbm, out_hbm, ids_v, hist_v, sem):
        wid = lax.axis_index("s") * NC + lax.axis_index("c")
        ids_dma = pltpu.async_copy(ids_hbm.at[pl.ds(wid * PER_W, PER_W)], ids_v, sem)

        zeros16 = jnp.zeros((LANES,), jnp.float32)

        def zero_chunk(j, _):
            for u in range(8):
                hist_v[pl.ds((j * 8 + u) * LANES, LANES)] = zeros16
            return 0

        lax.fori_loop(0, LANES * NUM_BINS // (8 * LANES), zero_chunk, 0)
        ids_dma.wait()

        lane_off = lax.iota(jnp.int32, LANES) * NUM_BINS
        ones = jnp.ones((LANES,), jnp.float32)

        def step(i, _):
            for u in range(8):
                v = ids_v[pl.ds((i * 8 + u) * LANES, LANES)]
                plsc.addupdate_scatter(hist_v, [lane_off + v], ones)
            return 0

        lax.fori_loop(0, CHUNKS // 8, step, 0)
        pltpu.sync_copy(hist_v, out_hbm.at[wid])

    return _sc_hist


def _tc_hist_body(ids_ref, out_ref):
    x = ids_ref[...]
    bins = lax.broadcasted_iota(jnp.int32, (1, 1, NUM_BINS), 2)
    eq = (x[:, :, None] == bins).astype(jnp.float32)
    partial = jnp.sum(eq, axis=(0, 1)).reshape(1, NUM_BINS)

    @pl.when(pl.program_id(0) == 0)
    def _init():
        out_ref[...] = partial

    @pl.when(pl.program_id(0) != 0)
    def _acc():
        out_ref[...] += partial


_tc_hist = pl.pallas_call(
    _tc_hist_body,
    grid=(TC_ROWS // TC_BLOCK,),
    in_specs=[
        pl.BlockSpec((TC_BLOCK, 128), lambda i: (TC_ROWS // TC_BLOCK + i, 0))
    ],
    out_specs=pl.BlockSpec((1, NUM_BINS), lambda i: (0, 0)),
    out_shape=jax.ShapeDtypeStruct((1, NUM_BINS), jnp.float32),
    compiler_params=pltpu.CompilerParams(dimension_semantics=("arbitrary",)),
)


def _tc_fold_body(hist_ref, counts_tc_ref, table_ref, out_ref):
    counts = (
        jnp.dot(
            jnp.ones((1, NW * LANES), jnp.float32),
            hist_ref[...],
            preferred_element_type=jnp.float32,
        )
        + counts_tc_ref[...]
    )
    sentence = jnp.dot(
        counts, table_ref[...], preferred_element_type=jnp.float32
    ) * (1.0 / NUM_IDS)
    x = sentence
    width = PACKET
    while width > 4:
        half = width // 2
        x = jnp.tanh(x[:, :half] + x[:, half:width])
        width = half
    out_ref[...] = x


_tc_fold = pl.pallas_call(
    _tc_fold_body,
    out_shape=jax.ShapeDtypeStruct((1, 4), jnp.float32),
)


def kernel(ids, char_embed):
    ids = ids.astype(jnp.int32)
    hist_sc = _make_sc_hist()(ids)
    counts_tc = _tc_hist(ids.reshape(2 * TC_ROWS, 128))
    return _tc_fold(
        hist_sc.reshape(NW * LANES, NUM_BINS), counts_tc, char_embed
    )


# speedup vs baseline: 1.3029x; 1.3029x over previous
import functools

import jax
import jax.numpy as jnp
from jax import lax
from jax.experimental import pallas as pl
from jax.experimental.pallas import tpu as pltpu
from jax.experimental.pallas import tpu_sc as plsc

NUM_IDS = 131072
NUM_BINS = 256
PACKET = 512
NC = 2
NS = 16
LANES = 16
NW = NC * NS
PER_W = NUM_IDS // NW
CHUNKS = PER_W // LANES


@functools.lru_cache(maxsize=None)
def _make_sc_hist():
    mesh = plsc.VectorSubcoreMesh(
        core_axis_name="c", subcore_axis_name="s", num_cores=NC, num_subcores=NS
    )

    @functools.partial(
        pl.kernel,
        out_type=jax.ShapeDtypeStruct((NW, LANES * NUM_BINS), jnp.float32),
        mesh=mesh,
        scratch_types=[
            pltpu.VMEM((PER_W,), jnp.int32),
            pltpu.VMEM((LANES * NUM_BINS,), jnp.float32),
            pltpu.SemaphoreType.DMA,
        ],
        compiler_params=pltpu.CompilerParams(needs_layout_passes=False),
    )
    def _sc_hist(ids_hbm, out_hbm, ids_v, hist_v, sem):
        wid = lax.axis_index("s") * NC + lax.axis_index("c")
        ids_dma = pltpu.async_copy(ids_hbm.at[pl.ds(wid * PER_W, PER_W)], ids_v, sem)

        zeros16 = jnp.zeros((LANES,), jnp.float32)

        @plsc.parallel_loop(0, LANES * NUM_BINS, LANES, unroll=8)
        def _zero(j):
            hist_v[pl.ds(j, LANES)] = zeros16

        ids_dma.wait()

        lane_off = lax.iota(jnp.int32, LANES) * NUM_BINS
        ones = jnp.ones((LANES,), jnp.float32)

        @plsc.parallel_loop(0, PER_W, LANES, unroll=8)
        def _scatter(i):
            v = ids_v[pl.ds(i, LANES)]
            plsc.addupdate_scatter(hist_v, [lane_off + v], ones)

        pltpu.sync_copy(hist_v, out_hbm.at[wid])

    return _sc_hist


def _tc_body(hist_ref, table_ref, out_ref):
    counts = jnp.sum(hist_ref[...], axis=0).reshape(1, NUM_BINS)
    sentence = jnp.dot(
        counts, table_ref[...], preferred_element_type=jnp.float32
    ) * (1.0 / NUM_IDS)
    x = sentence
    width = PACKET
    while width > 4:
        half = width // 2
        x = jnp.tanh(x[:, :half] + x[:, half:width])
        width = half
    out_ref[...] = x


_tc_fold = pl.pallas_call(
    _tc_body,
    out_shape=jax.ShapeDtypeStruct((1, 4), jnp.float32),
)


def kernel(ids, char_embed):
    ids = ids.astype(jnp.int32)
    hist = _make_sc_hist()(ids)
    return _tc_fold(hist.reshape(NW * LANES, NUM_BINS), char_embed)
